# baseline (speedup 1.0000x reference)
import functools

import jax
import jax.numpy as jnp
from jax import lax
from jax.experimental import pallas as pl
from jax.experimental.pallas import tpu as pltpu
from jax.experimental.pallas import tpu_sc as plsc

TOKENS = 8192
DIM = 4096
NUM_EXPERTS = 64
TOPK = 8
EPSILON = 1e-06

NC = 2
NS = 16
L = 16
NW = NC * NS
EV = NUM_EXPERTS // L

CHUNKS = 1
CTOK = TOKENS // CHUNKS
BT = min(1024, CTOK)
TPW = CTOK // NW
NGRP = TPW // L
SLAB = TPW * NUM_EXPERTS


NK = 1
KB = DIM // NK


def _gate_body(x_ref, w_ref, b_ref, gate_ref, acc_ref):
    k = pl.program_id(1)
    part = lax.dot_general(
        w_ref[...], x_ref[...],
        dimension_numbers=(((1,), (1,)), ((), ())),
        preferred_element_type=jnp.float32,
    )

    @pl.when(k == 0)
    def _init():
        acc_ref[...] = part + b_ref[...]

    @pl.when(k == NK - 1)
    def _fin():
        logits = acc_ref[...] + part if NK > 1 else acc_ref[...]
        m = jnp.max(logits, axis=0, keepdims=True)
        e = jnp.exp(logits - m)
        g = e / jnp.sum(e, axis=0, keepdims=True)
        for j in range(BT // TPW):
            gate_ref[j, :, :] = g[:, j * TPW:(j + 1) * TPW]


def _tc_gate(x, W, b2, c):
    nb = CTOK // BT
    off = c * nb
    return pl.pallas_call(
        _gate_body,
        grid=(nb, NK),
        in_specs=[
            pl.BlockSpec((BT, KB), lambda i, k, _o=off: (_o + i, k)),
            pl.BlockSpec((NUM_EXPERTS, KB), lambda i, k: (0, k)),
            pl.BlockSpec((NUM_EXPERTS, 1), lambda i, k: (0, 0)),
        ],
        out_specs=pl.BlockSpec((BT // TPW, NUM_EXPERTS, TPW),
                               lambda i, k: (i, 0, 0)),
        out_shape=jax.ShapeDtypeStruct((NW, NUM_EXPERTS, TPW), jnp.float32),
        scratch_shapes=[pltpu.VMEM((NUM_EXPERTS, BT), jnp.float32)],
        compiler_params=pltpu.CompilerParams(
            dimension_semantics=("arbitrary", "arbitrary"),
        ),
    )(x, W, b2)


_SORT8 = [(0, 1), (2, 3), (4, 5), (6, 7), (0, 2), (1, 3), (4, 6), (5, 7),
          (1, 2), (5, 6), (0, 4), (1, 5), (2, 6), (3, 7), (2, 4), (3, 5),
          (1, 2), (3, 4), (5, 6)]
_BITONIC8 = [(0, 4), (1, 5), (2, 6), (3, 7), (0, 2), (1, 3), (4, 6), (5, 7),
             (0, 1), (2, 3), (4, 5), (6, 7)]


def _cas_net(vs, net):
    vs = list(vs)
    for a, b in net:
        hi = jnp.maximum(vs[a], vs[b])
        lo = jnp.minimum(vs[a], vs[b])
        vs[a], vs[b] = hi, lo
    return vs


def _topk_threshold(load):
    top = _cas_net([load(e) for e in range(8)], _SORT8)
    for blk in range(1, 8):
        srt = _cas_net([load(blk * 8 + i) for i in range(8)], _SORT8)
        merged = [jnp.maximum(top[i], srt[7 - i]) for i in range(8)]
        if blk < 7:
            top = _cas_net(merged, _BITONIC8)
        else:
            while len(merged) > 1:
                merged = [jnp.minimum(merged[a], merged[a + 1])
                          for a in range(0, len(merged), 2)]
            return merged[0]


def _route_body(gate_hbm, masked_hbm, part_hbm, buf_in, buf_out, buf_acc,
                sem):
    wid = lax.axis_index("s") * NC + lax.axis_index("c")
    pltpu.sync_copy(gate_hbm.at[wid], buf_in)

    def group(g, carry):
        def load(e):
            return buf_in[pl.ds(e * TPW + g * L, L)]

        thr = _topk_threshold(load)
        rows = g * L + lax.iota(jnp.int32, L)
        for e in range(NUM_EXPERTS):
            ve = load(e)
            me = jnp.where(ve >= thr, ve, 0.0)
            plsc.store_scatter(buf_out, [rows * NUM_EXPERTS + e], me)
        return carry

    lax.fori_loop(0, NGRP, group, 0)

    cp = pltpu.async_copy(buf_out, masked_hbm.at[wid], sem)

    zero = jnp.zeros((L,), jnp.float32)

    def csum(t, acc):
        return tuple(acc[j] + buf_out[pl.ds(t * NUM_EXPERTS + L * j, L)]
                     for j in range(EV))

    acc = lax.fori_loop(0, TPW, csum, (zero,) * EV)
    for j in range(EV):
        buf_acc[pl.ds(L * j, L)] = acc[j]
    pltpu.sync_copy(buf_acc,
                    part_hbm.at[pl.ds(wid * NUM_EXPERTS, NUM_EXPERTS)])
    cp.wait()


def _norm_body(*refs):
    masked_refs = refs[:CHUNKS]
    part_refs = refs[CHUNKS:2 * CHUNKS]
    out_hbm = refs[2 * CHUNKS]
    buf_m, buf_p, buf_o = refs[2 * CHUNKS + 1:]
    wid = lax.axis_index("s") * NC + lax.axis_index("c")
    for c in range(CHUNKS):
        pltpu.sync_copy(part_refs[c], buf_p.at[pl.ds(c * NW * NUM_EXPERTS,
                                                     NW * NUM_EXPERTS)])
    zero = jnp.zeros((L,), jnp.float32)

    def red(w2, acc):
        return tuple(acc[j] + buf_p[pl.ds(w2 * NUM_EXPERTS + L * j, L)]
                     for j in range(EV))

    tot = lax.fori_loop(0, CHUNKS * NW, red, (zero,) * EV)
    inv = [1.0 / (tot[j] + EPSILON) for j in range(EV)]

    def row(t, carry):
        for j in range(EV):
            base = t * NUM_EXPERTS + L * j
            buf_o[pl.ds(base, L)] = buf_m[pl.ds(base, L)] * inv[j]
        return carry

    for c in range(CHUNKS):
        pltpu.sync_copy(masked_refs[c].at[wid], buf_m)
        lax.fori_loop(0, TPW, row, 0)
        pltpu.sync_copy(buf_o, out_hbm.at[c * NW + wid])


_sc_mesh = plsc.VectorSubcoreMesh(core_axis_name="c", subcore_axis_name="s")

_route = functools.partial(
    pl.kernel,
    out_type=[
        jax.ShapeDtypeStruct((NW, SLAB), jnp.float32),
        jax.ShapeDtypeStruct((NW * NUM_EXPERTS,), jnp.float32),
    ],
    mesh=_sc_mesh,
    scratch_types=[
        pltpu.VMEM((SLAB,), jnp.float32),
        pltpu.VMEM((SLAB,), jnp.float32),
        pltpu.VMEM((NUM_EXPERTS,), jnp.float32),
        pltpu.SemaphoreType.DMA,
    ],
    compiler_params=pltpu.CompilerParams(needs_layout_passes=False),
)(_route_body)

_norm = functools.partial(
    pl.kernel,
    out_type=jax.ShapeDtypeStruct((CHUNKS * NW, SLAB), jnp.float32),
    mesh=_sc_mesh,
    scratch_types=[
        pltpu.VMEM((SLAB,), jnp.float32),
        pltpu.VMEM((CHUNKS * NW * NUM_EXPERTS,), jnp.float32),
        pltpu.VMEM((SLAB,), jnp.float32),
    ],
)(_norm_body)


@jax.jit
def kernel(x, W, b):
    b2 = b.reshape(NUM_EXPERTS, 1)
    gates = [None] * CHUNKS
    masked = [None] * CHUNKS
    parts = [None] * CHUNKS
    gates[0] = _tc_gate(x, W, b2, 0)
    for c in range(CHUNKS):
        if c + 1 < CHUNKS:
            gates[c + 1] = _tc_gate(x, W, b2, c + 1)
        masked[c], parts[c] = _route(gates[c].reshape(NW, SLAB))
    out = _norm(*masked, *parts)
    return out.reshape(TOKENS, NUM_EXPERTS)


# speedup vs baseline: 1.0025x; 1.0025x over previous
import functools

import jax
import jax.numpy as jnp
from jax import lax
from jax.experimental import pallas as pl
from jax.experimental.pallas import tpu as pltpu
from jax.experimental.pallas import tpu_sc as plsc

TOKENS = 8192
DIM = 4096
NUM_EXPERTS = 64
TOPK = 8
EPSILON = 1e-06

NC = 2
NS = 16
L = 16
NW = NC * NS
EV = NUM_EXPERTS // L

CHUNKS = 1
CTOK = TOKENS // CHUNKS
BT = min(1024, CTOK)
TPW = CTOK // NW
NGRP = TPW // L
SLAB = TPW * NUM_EXPERTS


NK = 1
KB = DIM // NK


def _gate_body(x_ref, w_ref, b_ref, gate_ref, acc_ref):
    k = pl.program_id(1)
    part = lax.dot_general(
        w_ref[...], x_ref[...],
        dimension_numbers=(((1,), (1,)), ((), ())),
        preferred_element_type=jnp.float32,
    )

    @pl.when(k == 0)
    def _init():
        acc_ref[...] = part + b_ref[...]

    @pl.when(k == NK - 1)
    def _fin():
        logits = acc_ref[...] + part if NK > 1 else acc_ref[...]
        m = jnp.max(logits, axis=0, keepdims=True)
        e = jnp.exp(logits - m)
        g = e / jnp.sum(e, axis=0, keepdims=True)
        for j in range(BT // TPW):
            gate_ref[j, :, :] = g[:, j * TPW:(j + 1) * TPW]


def _tc_gate(x, W, b2, c):
    nb = CTOK // BT
    off = c * nb
    return pl.pallas_call(
        _gate_body,
        grid=(nb, NK),
        in_specs=[
            pl.BlockSpec((BT, KB), lambda i, k, _o=off: (_o + i, k)),
            pl.BlockSpec((NUM_EXPERTS, KB), lambda i, k: (0, k)),
            pl.BlockSpec((NUM_EXPERTS, 1), lambda i, k: (0, 0)),
        ],
        out_specs=pl.BlockSpec((BT // TPW, NUM_EXPERTS, TPW),
                               lambda i, k: (i, 0, 0)),
        out_shape=jax.ShapeDtypeStruct((NW, NUM_EXPERTS, TPW), jnp.float32),
        scratch_shapes=[pltpu.VMEM((NUM_EXPERTS, BT), jnp.float32)],
        compiler_params=pltpu.CompilerParams(
            dimension_semantics=("arbitrary", "arbitrary"),
        ),
    )(x, W, b2)


_SORT8 = [(0, 1), (2, 3), (4, 5), (6, 7), (0, 2), (1, 3), (4, 6), (5, 7),
          (1, 2), (5, 6), (0, 4), (1, 5), (2, 6), (3, 7), (2, 4), (3, 5),
          (1, 2), (3, 4), (5, 6)]
_BITONIC8 = [(0, 4), (1, 5), (2, 6), (3, 7), (0, 2), (1, 3), (4, 6), (5, 7),
             (0, 1), (2, 3), (4, 5), (6, 7)]


def _cas_net(vs, net):
    vs = list(vs)
    for a, b in net:
        hi = jnp.maximum(vs[a], vs[b])
        lo = jnp.minimum(vs[a], vs[b])
        vs[a], vs[b] = hi, lo
    return vs


def _topk_threshold(load):
    top = _cas_net([load(e) for e in range(8)], _SORT8)
    for blk in range(1, 8):
        srt = _cas_net([load(blk * 8 + i) for i in range(8)], _SORT8)
        merged = [jnp.maximum(top[i], srt[7 - i]) for i in range(8)]
        if blk < 7:
            top = _cas_net(merged, _BITONIC8)
        else:
            while len(merged) > 1:
                merged = [jnp.minimum(merged[a], merged[a + 1])
                          for a in range(0, len(merged), 2)]
            return merged[0]


def _route_body(gate_hbm, masked_hbm, part_hbm, buf_in, buf_out, buf_acc):
    wid = lax.axis_index("s") * NC + lax.axis_index("c")
    pltpu.sync_copy(gate_hbm.at[wid], buf_in)

    def group(g, carry):
        def load(e):
            return buf_in[pl.ds(e * TPW + g * L, L)]

        thr = _topk_threshold(load)
        rows = g * L + lax.iota(jnp.int32, L)
        for e in range(NUM_EXPERTS):
            ve = load(e)
            me = jnp.where(ve >= thr, ve, 0.0)
            plsc.store_scatter(buf_out, [rows * NUM_EXPERTS + e], me)
        return carry

    lax.fori_loop(0, NGRP, group, 0)

    zero = jnp.zeros((L,), jnp.float32)

    def csum(t, acc):
        return tuple(acc[j] + buf_out[pl.ds(t * NUM_EXPERTS + L * j, L)]
                     for j in range(EV))

    acc = lax.fori_loop(0, TPW, csum, (zero,) * EV)
    for j in range(EV):
        buf_acc[pl.ds(L * j, L)] = acc[j]
    pltpu.sync_copy(buf_out, masked_hbm.at[wid])
    pltpu.sync_copy(buf_acc,
                    part_hbm.at[pl.ds(wid * NUM_EXPERTS, NUM_EXPERTS)])


def _norm_body(*refs):
    masked_refs = refs[:CHUNKS]
    part_refs = refs[CHUNKS:2 * CHUNKS]
    out_hbm = refs[2 * CHUNKS]
    buf_m, buf_p, buf_o = refs[2 * CHUNKS + 1:]
    wid = lax.axis_index("s") * NC + lax.axis_index("c")
    for c in range(CHUNKS):
        pltpu.sync_copy(part_refs[c], buf_p.at[pl.ds(c * NW * NUM_EXPERTS,
                                                     NW * NUM_EXPERTS)])
    zero = jnp.zeros((L,), jnp.float32)

    def red(w2, acc):
        return tuple(acc[j] + buf_p[pl.ds(w2 * NUM_EXPERTS + L * j, L)]
                     for j in range(EV))

    tot = lax.fori_loop(0, CHUNKS * NW, red, (zero,) * EV)
    inv = [1.0 / (tot[j] + EPSILON) for j in range(EV)]

    def row(t, carry):
        for j in range(EV):
            base = t * NUM_EXPERTS + L * j
            buf_o[pl.ds(base, L)] = buf_m[pl.ds(base, L)] * inv[j]
        return carry

    for c in range(CHUNKS):
        pltpu.sync_copy(masked_refs[c].at[wid], buf_m)
        lax.fori_loop(0, TPW, row, 0)
        pltpu.sync_copy(buf_o, out_hbm.at[c * NW + wid])


_sc_mesh = plsc.VectorSubcoreMesh(core_axis_name="c", subcore_axis_name="s")

_route = functools.partial(
    pl.kernel,
    out_type=[
        jax.ShapeDtypeStruct((NW, SLAB), jnp.float32),
        jax.ShapeDtypeStruct((NW * NUM_EXPERTS,), jnp.float32),
    ],
    mesh=_sc_mesh,
    scratch_types=[
        pltpu.VMEM((SLAB,), jnp.float32),
        pltpu.VMEM((SLAB,), jnp.float32),
        pltpu.VMEM((NUM_EXPERTS,), jnp.float32),
    ],
    compiler_params=pltpu.CompilerParams(needs_layout_passes=False),
)(_route_body)

_norm = functools.partial(
    pl.kernel,
    out_type=jax.ShapeDtypeStruct((CHUNKS * NW, SLAB), jnp.float32),
    mesh=_sc_mesh,
    scratch_types=[
        pltpu.VMEM((SLAB,), jnp.float32),
        pltpu.VMEM((CHUNKS * NW * NUM_EXPERTS,), jnp.float32),
        pltpu.VMEM((SLAB,), jnp.float32),
    ],
)(_norm_body)


@jax.jit
def kernel(x, W, b):
    b2 = b.reshape(NUM_EXPERTS, 1)
    gates = [None] * CHUNKS
    masked = [None] * CHUNKS
    parts = [None] * CHUNKS
    gates[0] = _tc_gate(x, W, b2, 0)
    for c in range(CHUNKS):
        if c + 1 < CHUNKS:
            gates[c + 1] = _tc_gate(x, W, b2, c + 1)
        masked[c], parts[c] = _route(gates[c].reshape(NW, SLAB))
    out = _norm(*masked, *parts)
    return out.reshape(TOKENS, NUM_EXPERTS)
